# baseline (speedup 1.0000x reference)
import functools

import jax
import jax.numpy as jnp
from jax import lax
from jax.experimental import pallas as pl
from jax.experimental.pallas import tpu as pltpu
from jax.experimental.pallas import tpu_sc as plsc

N = 10000
D = 128
E = 320000
NC = 2
NS = 16
NW = NC * NS
CHUNK = 128
CHPT = 80
EPAD = NW * CHPT * CHUNK
CHPT0 = EPAD // (NS * CHUNK)
ACC_ROWS = N + 112
RPT = ACC_ROWS // NS
RB = 2000


def _mesh():
    return plsc.VectorSubcoreMesh(
        core_axis_name="c", subcore_axis_name="s", num_cores=NC, num_subcores=NS
    )


@functools.partial(
    pl.kernel,
    out_type=jax.ShapeDtypeStruct((NW * ACC_ROWS,), jnp.float32),
    mesh=_mesh(),
    scratch_types=[
        pltpu.VMEM((ACC_ROWS,), jnp.float32),
        pltpu.VMEM((CHPT * CHUNK,), jnp.int32),
        pltpu.SemaphoreType.DMA,
    ],
    compiler_params=pltpu.CompilerParams(needs_layout_passes=False),
)
def _sc_deg(row_hbm, out_hbm, deg_v, idx_v, sem):
    c = lax.axis_index("c")
    s = lax.axis_index("s")
    wid = s * NC + c
    zeros16 = jnp.zeros((16,), jnp.float32)
    ones16 = jnp.ones((16,), jnp.float32)

    idx_dma = pltpu.async_copy(
        row_hbm.at[pl.ds(wid * (CHPT * CHUNK), CHPT * CHUNK)], idx_v, sem
    )

    @pl.loop(0, ACC_ROWS // 16)
    def _zero(i):
        deg_v[pl.ds(i * 16, 16)] = zeros16

    idx_dma.wait()

    @pl.loop(0, CHPT * CHUNK // 16)
    def _groups(i):
        idx16 = idx_v[pl.ds(i * 16, 16)]
        plsc.addupdate_scatter(deg_v, [idx16], ones16)

    pltpu.sync_copy(deg_v, out_hbm.at[pl.ds(wid * ACC_ROWS, ACC_ROWS)])


@functools.partial(
    pl.kernel,
    out_type=jax.ShapeDtypeStruct((ACC_ROWS, D), jnp.float32),
    mesh=_mesh(),
    scratch_types=[
        pltpu.VMEM_SHARED((ACC_ROWS, D), jnp.float32),
        [pltpu.VMEM((CHUNK, D), jnp.float32) for _ in range(2)],
        [pltpu.VMEM((CHUNK,), jnp.int32) for _ in range(2)],
        [pltpu.VMEM((CHUNK,), jnp.int32) for _ in range(2)],
        [pltpu.SemaphoreType.DMA for _ in range(2)],
        [pltpu.SemaphoreType.DMA for _ in range(2)],
        [pltpu.SemaphoreType.DMA for _ in range(2)],
        [pltpu.SemaphoreType.DMA for _ in range(2)],
    ],
    compiler_params=pltpu.CompilerParams(needs_layout_passes=False),
)
def _sc_gather_scatter(
    g_hbm, row_hbm, col_hbm, out_hbm, acc, bufs, cis, ris, gsems, ssems, csems, rsems
):
    c = lax.axis_index("c")
    s = lax.axis_index("s")

    @pl.when(c == 0)
    def _body():
        zeros16 = jnp.zeros((16,), jnp.float32)
        cbase = s * CHPT0
        T = CHPT0
        r0 = s * RPT

        with jax.named_scope("zero_acc"):
            @pl.loop(0, CHUNK)
            def _zb(i):
                for j in range(D // 16):
                    bufs[0][i, pl.ds(j * 16, 16)] = zeros16

            off = 0
            while off < RPT:
                take = min(CHUNK, RPT - off)
                pltpu.sync_copy(
                    bufs[0].at[pl.ds(0, take)], acc.at[pl.ds(r0 + off, take)]
                )
                off += take
            plsc.subcore_barrier()

        def issue_cidx(t, k):
            pltpu.async_copy(
                col_hbm.at[pl.ds((cbase + t) * CHUNK, CHUNK)], cis[k], csems[k]
            )

        def wait_cidx(t, k):
            pltpu.make_async_copy(
                col_hbm.at[pl.ds((cbase + t) * CHUNK, CHUNK)], cis[k], csems[k]
            ).wait()

        def issue_ridx(t, k):
            pltpu.async_copy(
                row_hbm.at[pl.ds((cbase + t) * CHUNK, CHUNK)], ris[k], rsems[k]
            )

        def wait_ridx(t, k):
            pltpu.make_async_copy(
                row_hbm.at[pl.ds((cbase + t) * CHUNK, CHUNK)], ris[k], rsems[k]
            ).wait()

        def issue_gather(k):
            pltpu.async_copy(g_hbm.at[cis[k]], bufs[k], gsems[k])

        def wait_gather(k):
            pltpu.make_async_copy(g_hbm.at[cis[k]], bufs[k], gsems[k]).wait()

        def issue_scatter(k):
            pltpu.async_copy(bufs[k], acc.at[ris[k]], ssems[k], add=True)

        def wait_scatter(k):
            pltpu.make_async_copy(bufs[k], acc.at[ris[k]], ssems[k]).wait()

        with jax.named_scope("edge_loop"):
            issue_cidx(0, 0)

            @pl.loop(0, T // 2)
            def _steps(q):
                t0 = 2 * q
                for k in range(2):
                    t = t0 + k
                    k1 = 1 - k

                    @pl.when(t >= 1)
                    def _():
                        wait_ridx(t - 1, k1)
                        wait_gather(k1)
                        issue_scatter(k1)

                    @pl.when(t + 1 < T)
                    def _():
                        issue_cidx(t + 1, k1)

                    @pl.when(t >= 2)
                    def _():
                        wait_scatter(k)

                    issue_ridx(t, k)
                    wait_cidx(t, k)
                    issue_gather(k)

            wait_ridx(T - 1, 1)
            wait_gather(1)
            issue_scatter(1)
            wait_scatter(0)
            wait_scatter(1)

        with jax.named_scope("writeout"):
            plsc.subcore_barrier()
            pltpu.sync_copy(acc.at[pl.ds(r0, RPT)], out_hbm.at[pl.ds(r0, RPT)])


def _tc1_body(deg_ref, x_ref, w_ref, dinv_ref, g1_ref, out0_ref):
    deg = jnp.sum(deg_ref[...], axis=1)
    dinv = jnp.where(deg > 0, lax.rsqrt(jnp.where(deg > 0, deg, 1.0)), 0.0)
    d = dinv[:, None]
    dinv_ref[...] = d
    xv = x_ref[...]
    g1_ref[...] = d * xv
    out0_ref[...] = jnp.dot(xv, w_ref[...], preferred_element_type=jnp.float32)


def _tc1(degp, x, w0):
    return pl.pallas_call(
        _tc1_body,
        grid=(N // RB,),
        in_specs=[
            pl.BlockSpec((RB, NW), lambda i: (i, 0)),
            pl.BlockSpec((RB, D), lambda i: (i, 0)),
            pl.BlockSpec((D, D), lambda i: (0, 0)),
        ],
        out_specs=[
            pl.BlockSpec((RB, 1), lambda i: (i, 0)),
            pl.BlockSpec((RB, D), lambda i: (i, 0)),
            pl.BlockSpec((RB, D), lambda i: (i, 0)),
        ],
        out_shape=[
            jax.ShapeDtypeStruct((N, 1), jnp.float32),
            jax.ShapeDtypeStruct((N, D), jnp.float32),
            jax.ShapeDtypeStruct((N, D), jnp.float32),
        ],
    )(degp, x, w0)


def _tc2_body(s_ref, dinv_ref, out0_ref, w_ref, out1_ref, g2_ref):
    d = dinv_ref[...]
    t = -d * s_ref[...]
    out1_ref[...] = out0_ref[...] + jnp.dot(
        t, w_ref[...], preferred_element_type=jnp.float32
    )
    g2_ref[...] = d * t


def _tc2(s1, dinv, out0, w1):
    row = pl.BlockSpec((RB, D), lambda i: (i, 0))
    return pl.pallas_call(
        _tc2_body,
        grid=(N // RB,),
        in_specs=[
            row,
            pl.BlockSpec((RB, 1), lambda i: (i, 0)),
            row,
            pl.BlockSpec((D, D), lambda i: (0, 0)),
        ],
        out_specs=[row, row],
        out_shape=[
            jax.ShapeDtypeStruct((N, D), jnp.float32),
            jax.ShapeDtypeStruct((N, D), jnp.float32),
        ],
    )(s1, dinv, out0, w1)


def _tc3_body(s_ref, dinv_ref, x_ref, out1_ref, w_ref, bias_ref, y_ref):
    d = dinv_ref[...]
    xv = x_ref[...]
    tx2 = -2.0 * d * s_ref[...] - xv
    o = (
        out1_ref[...]
        + jnp.dot(tx2, w_ref[...], preferred_element_type=jnp.float32)
        + bias_ref[...]
    )
    y_ref[...] = jnp.maximum(o + xv, 0.0)


def _tc3(s2, dinv, x, out1, w2, bias):
    row = pl.BlockSpec((RB, D), lambda i: (i, 0))
    return pl.pallas_call(
        _tc3_body,
        grid=(N // RB,),
        in_specs=[
            row,
            pl.BlockSpec((RB, 1), lambda i: (i, 0)),
            row,
            row,
            pl.BlockSpec((D, D), lambda i: (0, 0)),
            pl.BlockSpec((1, D), lambda i: (0, 0)),
        ],
        out_specs=row,
        out_shape=jax.ShapeDtypeStruct((N, D), jnp.float32),
    )(s2, dinv, x, out1, w2, bias)


def kernel(x, edge_index, W, b):
    x = x.astype(jnp.float32)
    row = edge_index[0].astype(jnp.int32)
    col = edge_index[1].astype(jnp.int32)
    pad = jnp.full((EPAD - E,), N, jnp.int32)
    rowp = jnp.concatenate([row, pad])
    colp = jnp.concatenate([col, pad])
    zpad = jnp.zeros((ACC_ROWS - N, D), jnp.float32)

    degp = _sc_deg(rowp).reshape(NW, ACC_ROWS)
    dinv, g1, out0 = _tc1(degp[:, :N].T, x, W[0])
    s1 = _sc_gather_scatter(jnp.concatenate([g1, zpad]), rowp, colp)
    out1, g2 = _tc2(s1[:N], dinv, out0, W[1])
    s2 = _sc_gather_scatter(jnp.concatenate([g2, zpad]), rowp, colp)
    return _tc3(s2[:N], dinv, x, out1, W[2], b.reshape(1, D))


# speedup vs baseline: 3.8991x; 3.8991x over previous
import functools

import jax
import jax.numpy as jnp
from jax import lax
from jax.experimental import pallas as pl
from jax.experimental.pallas import tpu as pltpu
from jax.experimental.pallas import tpu_sc as plsc

N = 10000
D = 128
E = 320000
NC = 2
NS = 16
NW = NC * NS
CHUNK = 128
CHPT = 80
EPAD = NW * CHPT * CHUNK
CHPT0 = EPAD // (NS * CHUNK)
ACC_ROWS = N + 112
RPT = ACC_ROWS // NS
RB = 2000


def _mesh():
    return plsc.VectorSubcoreMesh(
        core_axis_name="c", subcore_axis_name="s", num_cores=NC, num_subcores=NS
    )


@functools.partial(
    pl.kernel,
    out_type=jax.ShapeDtypeStruct((NW * ACC_ROWS,), jnp.float32),
    mesh=_mesh(),
    scratch_types=[
        pltpu.VMEM((ACC_ROWS,), jnp.float32),
        pltpu.VMEM((CHPT * CHUNK,), jnp.int32),
        pltpu.SemaphoreType.DMA,
    ],
    compiler_params=pltpu.CompilerParams(needs_layout_passes=False),
)
def _sc_deg(row_hbm, out_hbm, deg_v, idx_v, sem):
    c = lax.axis_index("c")
    s = lax.axis_index("s")
    wid = s * NC + c
    zeros16 = jnp.zeros((16,), jnp.float32)
    ones16 = jnp.ones((16,), jnp.float32)

    idx_dma = pltpu.async_copy(
        row_hbm.at[pl.ds(wid * (CHPT * CHUNK), CHPT * CHUNK)], idx_v, sem
    )

    @pl.loop(0, ACC_ROWS // 16)
    def _zero(i):
        deg_v[pl.ds(i * 16, 16)] = zeros16

    idx_dma.wait()

    @pl.loop(0, CHPT * CHUNK // 16)
    def _groups(i):
        idx16 = idx_v[pl.ds(i * 16, 16)]
        plsc.addupdate_scatter(deg_v, [idx16], ones16)

    pltpu.sync_copy(deg_v, out_hbm.at[pl.ds(wid * ACC_ROWS, ACC_ROWS)])


@functools.partial(
    pl.kernel,
    out_type=jax.ShapeDtypeStruct((NC * ACC_ROWS, D), jnp.float32),
    mesh=_mesh(),
    scratch_types=[
        pltpu.VMEM_SHARED((ACC_ROWS, D), jnp.float32),
        [pltpu.VMEM((CHUNK, D), jnp.float32) for _ in range(2)],
        [pltpu.VMEM((CHUNK,), jnp.int32) for _ in range(2)],
        [pltpu.VMEM((CHUNK,), jnp.int32) for _ in range(2)],
        [pltpu.SemaphoreType.DMA for _ in range(2)],
        [pltpu.SemaphoreType.DMA for _ in range(2)],
        [pltpu.SemaphoreType.DMA for _ in range(2)],
        [pltpu.SemaphoreType.DMA for _ in range(2)],
    ],
    compiler_params=pltpu.CompilerParams(needs_layout_passes=False),
)
def _sc_gather_scatter(
    g_hbm, row_hbm, col_hbm, out_hbm, acc, bufs, cis, ris, gsems, ssems, csems, rsems
):
    c = lax.axis_index("c")
    s = lax.axis_index("s")
    wid = s * NC + c
    zeros16 = jnp.zeros((16,), jnp.float32)
    cbase = wid * CHPT
    T = CHPT
    r0 = s * RPT

    with jax.named_scope("zero_acc"):
        @pl.loop(0, CHUNK)
        def _zb(i):
            for j in range(D // 16):
                bufs[0][i, pl.ds(j * 16, 16)] = zeros16

        off = 0
        while off < RPT:
            take = min(CHUNK, RPT - off)
            pltpu.sync_copy(
                bufs[0].at[pl.ds(0, take)], acc.at[pl.ds(r0 + off, take)]
            )
            off += take
        plsc.subcore_barrier()

    def issue_cidx(t, k):
        pltpu.async_copy(
            col_hbm.at[pl.ds((cbase + t) * CHUNK, CHUNK)], cis[k], csems[k]
        )

    def wait_cidx(t, k):
        pltpu.make_async_copy(
            col_hbm.at[pl.ds((cbase + t) * CHUNK, CHUNK)], cis[k], csems[k]
        ).wait()

    def issue_ridx(t, k):
        pltpu.async_copy(
            row_hbm.at[pl.ds((cbase + t) * CHUNK, CHUNK)], ris[k], rsems[k]
        )

    def wait_ridx(t, k):
        pltpu.make_async_copy(
            row_hbm.at[pl.ds((cbase + t) * CHUNK, CHUNK)], ris[k], rsems[k]
        ).wait()

    def issue_gather(k):
        pltpu.async_copy(g_hbm.at[cis[k]], bufs[k], gsems[k])

    def wait_gather(k):
        pltpu.make_async_copy(g_hbm.at[cis[k]], bufs[k], gsems[k]).wait()

    def issue_scatter(k):
        pltpu.async_copy(bufs[k], acc.at[ris[k]], ssems[k], add=True)

    def wait_scatter(k):
        pltpu.make_async_copy(bufs[k], acc.at[ris[k]], ssems[k]).wait()

    with jax.named_scope("edge_loop"):
        issue_cidx(0, 0)

        @pl.loop(0, T // 2)
        def _steps(q):
            t0 = 2 * q
            for k in range(2):
                t = t0 + k
                k1 = 1 - k

                @pl.when(t >= 1)
                def _():
                    wait_ridx(t - 1, k1)
                    wait_gather(k1)
                    issue_scatter(k1)

                @pl.when(t + 1 < T)
                def _():
                    issue_cidx(t + 1, k1)

                @pl.when(t >= 2)
                def _():
                    wait_scatter(k)

                issue_ridx(t, k)
                wait_cidx(t, k)
                issue_gather(k)

        wait_ridx(T - 1, 1)
        wait_gather(1)
        issue_scatter(1)
        wait_scatter(0)
        wait_scatter(1)

    with jax.named_scope("writeout"):
        plsc.subcore_barrier()
        pltpu.sync_copy(
            acc.at[pl.ds(r0, RPT)], out_hbm.at[pl.ds(c * ACC_ROWS + r0, RPT)]
        )


def _tc1_body(deg_ref, x_ref, w_ref, dinv_ref, g1_ref, out0_ref):
    deg = jnp.sum(deg_ref[...], axis=1)
    dinv = jnp.where(deg > 0, lax.rsqrt(jnp.where(deg > 0, deg, 1.0)), 0.0)
    d = dinv[:, None]
    dinv_ref[...] = d
    xv = x_ref[...]
    g1_ref[...] = d * xv
    out0_ref[...] = jnp.dot(xv, w_ref[...], preferred_element_type=jnp.float32)


def _tc1(degp, x, w0):
    return pl.pallas_call(
        _tc1_body,
        grid=(N // RB,),
        in_specs=[
            pl.BlockSpec((RB, NW), lambda i: (i, 0)),
            pl.BlockSpec((RB, D), lambda i: (i, 0)),
            pl.BlockSpec((D, D), lambda i: (0, 0)),
        ],
        out_specs=[
            pl.BlockSpec((RB, 1), lambda i: (i, 0)),
            pl.BlockSpec((RB, D), lambda i: (i, 0)),
            pl.BlockSpec((RB, D), lambda i: (i, 0)),
        ],
        out_shape=[
            jax.ShapeDtypeStruct((N, 1), jnp.float32),
            jax.ShapeDtypeStruct((N, D), jnp.float32),
            jax.ShapeDtypeStruct((N, D), jnp.float32),
        ],
    )(degp, x, w0)


def _tc2_body(a_ref, b_ref, dinv_ref, out0_ref, w_ref, out1_ref, g2_ref):
    d = dinv_ref[...]
    t = -d * (a_ref[...] + b_ref[...])
    out1_ref[...] = out0_ref[...] + jnp.dot(
        t, w_ref[...], preferred_element_type=jnp.float32
    )
    g2_ref[...] = d * t


def _tc2(s1a, s1b, dinv, out0, w1):
    row = pl.BlockSpec((RB, D), lambda i: (i, 0))
    return pl.pallas_call(
        _tc2_body,
        grid=(N // RB,),
        in_specs=[
            row,
            row,
            pl.BlockSpec((RB, 1), lambda i: (i, 0)),
            row,
            pl.BlockSpec((D, D), lambda i: (0, 0)),
        ],
        out_specs=[row, row],
        out_shape=[
            jax.ShapeDtypeStruct((N, D), jnp.float32),
            jax.ShapeDtypeStruct((N, D), jnp.float32),
        ],
    )(s1a, s1b, dinv, out0, w1)


def _tc3_body(a_ref, b_ref, dinv_ref, x_ref, out1_ref, w_ref, bias_ref, y_ref):
    d = dinv_ref[...]
    xv = x_ref[...]
    tx2 = -2.0 * d * (a_ref[...] + b_ref[...]) - xv
    o = (
        out1_ref[...]
        + jnp.dot(tx2, w_ref[...], preferred_element_type=jnp.float32)
        + bias_ref[...]
    )
    y_ref[...] = jnp.maximum(o + xv, 0.0)


def _tc3(s2a, s2b, dinv, x, out1, w2, bias):
    row = pl.BlockSpec((RB, D), lambda i: (i, 0))
    return pl.pallas_call(
        _tc3_body,
        grid=(N // RB,),
        in_specs=[
            row,
            row,
            pl.BlockSpec((RB, 1), lambda i: (i, 0)),
            row,
            row,
            pl.BlockSpec((D, D), lambda i: (0, 0)),
            pl.BlockSpec((1, D), lambda i: (0, 0)),
        ],
        out_specs=row,
        out_shape=jax.ShapeDtypeStruct((N, D), jnp.float32),
    )(s2a, s2b, dinv, x, out1, w2, bias)


def kernel(x, edge_index, W, b):
    x = x.astype(jnp.float32)
    row = edge_index[0].astype(jnp.int32)
    col = edge_index[1].astype(jnp.int32)
    pad = N + (jnp.arange(EPAD - E, dtype=jnp.int32) % (ACC_ROWS - N))
    rowp = jnp.concatenate([row, pad])
    colp = jnp.concatenate([col, pad])
    zpad = jnp.zeros((ACC_ROWS - N, D), jnp.float32)

    degp = _sc_deg(rowp).reshape(NW, ACC_ROWS)
    dinv, g1, out0 = _tc1(degp[:, :N].T, x, W[0])
    s1 = _sc_gather_scatter(jnp.concatenate([g1, zpad]), rowp, colp)
    out1, g2 = _tc2(s1[:N], s1[ACC_ROWS : ACC_ROWS + N], dinv, out0, W[1])
    s2 = _sc_gather_scatter(jnp.concatenate([g2, zpad]), rowp, colp)
    return _tc3(
        s2[:N], s2[ACC_ROWS : ACC_ROWS + N], dinv, x, out1, W[2], b.reshape(1, D)
    )


# speedup vs baseline: 4.0194x; 1.0308x over previous
import functools

import jax
import jax.numpy as jnp
from jax import lax
from jax.experimental import pallas as pl
from jax.experimental.pallas import tpu as pltpu
from jax.experimental.pallas import tpu_sc as plsc

N = 10000
D = 128
E = 320000
NC = 2
NS = 16
NW = NC * NS
CHUNK = 128
CHPT = 80
EPAD = NW * CHPT * CHUNK
CHPT0 = EPAD // (NS * CHUNK)
ACC_ROWS = N + 112
RPT = ACC_ROWS // NS
RB = 2000


def _mesh():
    return plsc.VectorSubcoreMesh(
        core_axis_name="c", subcore_axis_name="s", num_cores=NC, num_subcores=NS
    )


@functools.partial(
    pl.kernel,
    out_type=jax.ShapeDtypeStruct((NW * ACC_ROWS,), jnp.float32),
    mesh=_mesh(),
    scratch_types=[
        pltpu.VMEM((ACC_ROWS,), jnp.float32),
        pltpu.VMEM((CHPT * CHUNK,), jnp.int32),
        pltpu.SemaphoreType.DMA,
    ],
    compiler_params=pltpu.CompilerParams(needs_layout_passes=False),
)
def _sc_deg(row_hbm, out_hbm, deg_v, idx_v, sem):
    c = lax.axis_index("c")
    s = lax.axis_index("s")
    wid = s * NC + c
    zeros16 = jnp.zeros((16,), jnp.float32)
    ones16 = jnp.ones((16,), jnp.float32)

    idx_dma = pltpu.async_copy(
        row_hbm.at[pl.ds(wid * (CHPT * CHUNK), CHPT * CHUNK)], idx_v, sem
    )

    @pl.loop(0, ACC_ROWS // 16)
    def _zero(i):
        deg_v[pl.ds(i * 16, 16)] = zeros16

    idx_dma.wait()

    @pl.loop(0, CHPT * CHUNK // 16)
    def _groups(i):
        idx16 = idx_v[pl.ds(i * 16, 16)]
        plsc.addupdate_scatter(deg_v, [idx16], ones16)

    pltpu.sync_copy(deg_v, out_hbm.at[pl.ds(wid * ACC_ROWS, ACC_ROWS)])


@functools.partial(
    pl.kernel,
    out_type=[
        jax.ShapeDtypeStruct((ACC_ROWS, D), jnp.float32),
        jax.ShapeDtypeStruct((ACC_ROWS, D), jnp.float32),
    ],
    mesh=_mesh(),
    scratch_types=[
        pltpu.VMEM_SHARED((ACC_ROWS, D), jnp.float32),
        [pltpu.VMEM((CHUNK, D), jnp.float32) for _ in range(2)],
        [pltpu.VMEM((CHUNK,), jnp.int32) for _ in range(2)],
        [pltpu.VMEM((CHUNK,), jnp.int32) for _ in range(2)],
        [pltpu.SemaphoreType.DMA for _ in range(2)],
        [pltpu.SemaphoreType.DMA for _ in range(2)],
        [pltpu.SemaphoreType.DMA for _ in range(2)],
        [pltpu.SemaphoreType.DMA for _ in range(2)],
    ],
    compiler_params=pltpu.CompilerParams(needs_layout_passes=False),
)
def _sc_gather_scatter(
    g_hbm, row_hbm, col_hbm, out0_hbm, out1_hbm, acc, bufs, cis, ris, gsems, ssems, csems, rsems
):
    c = lax.axis_index("c")
    s = lax.axis_index("s")
    wid = s * NC + c
    zeros16 = jnp.zeros((16,), jnp.float32)
    cbase = wid * CHPT
    T = CHPT
    r0 = s * RPT

    with jax.named_scope("zero_acc"):
        @pl.loop(0, CHUNK)
        def _zb(i):
            for j in range(D // 16):
                bufs[0][i, pl.ds(j * 16, 16)] = zeros16

        off = 0
        while off < RPT:
            take = min(CHUNK, RPT - off)
            pltpu.sync_copy(
                bufs[0].at[pl.ds(0, take)], acc.at[pl.ds(r0 + off, take)]
            )
            off += take
        plsc.subcore_barrier()

    def issue_cidx(t, k):
        pltpu.async_copy(
            col_hbm.at[pl.ds((cbase + t) * CHUNK, CHUNK)], cis[k], csems[k]
        )

    def wait_cidx(t, k):
        pltpu.make_async_copy(
            col_hbm.at[pl.ds((cbase + t) * CHUNK, CHUNK)], cis[k], csems[k]
        ).wait()

    def issue_ridx(t, k):
        pltpu.async_copy(
            row_hbm.at[pl.ds((cbase + t) * CHUNK, CHUNK)], ris[k], rsems[k]
        )

    def wait_ridx(t, k):
        pltpu.make_async_copy(
            row_hbm.at[pl.ds((cbase + t) * CHUNK, CHUNK)], ris[k], rsems[k]
        ).wait()

    def issue_gather(k):
        pltpu.async_copy(g_hbm.at[cis[k]], bufs[k], gsems[k])

    def wait_gather(k):
        pltpu.make_async_copy(g_hbm.at[cis[k]], bufs[k], gsems[k]).wait()

    def issue_scatter(k):
        pltpu.async_copy(bufs[k], acc.at[ris[k]], ssems[k], add=True)

    def wait_scatter(k):
        pltpu.make_async_copy(bufs[k], acc.at[ris[k]], ssems[k]).wait()

    with jax.named_scope("edge_loop"):
        issue_cidx(0, 0)

        @pl.loop(0, T // 2)
        def _steps(q):
            t0 = 2 * q
            for k in range(2):
                t = t0 + k
                k1 = 1 - k

                @pl.when(t >= 1)
                def _():
                    wait_ridx(t - 1, k1)
                    wait_gather(k1)
                    issue_scatter(k1)

                @pl.when(t + 1 < T)
                def _():
                    issue_cidx(t + 1, k1)

                @pl.when(t >= 2)
                def _():
                    wait_scatter(k)

                issue_ridx(t, k)
                wait_cidx(t, k)
                issue_gather(k)

        wait_ridx(T - 1, 1)
        wait_gather(1)
        issue_scatter(1)
        wait_scatter(0)
        wait_scatter(1)

    with jax.named_scope("writeout"):
        plsc.subcore_barrier()

        @pl.when(c == 0)
        def _w0():
            pltpu.sync_copy(acc.at[pl.ds(r0, RPT)], out0_hbm.at[pl.ds(r0, RPT)])

        @pl.when(c == 1)
        def _w1():
            pltpu.sync_copy(acc.at[pl.ds(r0, RPT)], out1_hbm.at[pl.ds(r0, RPT)])


RB2 = ACC_ROWS // 16


def _tc0_body(x_ref, w_ref, out0_ref):
    out0_ref[...] = jnp.dot(
        x_ref[...], w_ref[...], preferred_element_type=jnp.float32
    )


def _tc0(xp, w0):
    row = pl.BlockSpec((RB2, D), lambda i: (i, 0))
    return pl.pallas_call(
        _tc0_body,
        grid=(ACC_ROWS // RB2,),
        in_specs=[row, pl.BlockSpec((D, D), lambda i: (0, 0))],
        out_specs=row,
        out_shape=jax.ShapeDtypeStruct((ACC_ROWS, D), jnp.float32),
    )(xp, w0)


def _tc1_body(deg_ref, x_ref, dinv_ref, g1_ref):
    deg = jnp.sum(deg_ref[...], axis=0)
    dinv = jnp.where(deg > 0, lax.rsqrt(jnp.where(deg > 0, deg, 1.0)), 0.0)
    d = dinv[:, None]
    dinv_ref[...] = d
    g1_ref[...] = d * x_ref[...]


def _tc1(degp, xp):
    return pl.pallas_call(
        _tc1_body,
        grid=(1,),
        in_specs=[
            pl.BlockSpec((NW, ACC_ROWS), lambda i: (0, 0)),
            pl.BlockSpec((ACC_ROWS, D), lambda i: (0, 0)),
        ],
        out_specs=[
            pl.BlockSpec((ACC_ROWS, 1), lambda i: (0, 0)),
            pl.BlockSpec((ACC_ROWS, D), lambda i: (0, 0)),
        ],
        out_shape=[
            jax.ShapeDtypeStruct((ACC_ROWS, 1), jnp.float32),
            jax.ShapeDtypeStruct((ACC_ROWS, D), jnp.float32),
        ],
    )(degp, xp)


def _tc2_body(a_ref, b_ref, dinv_ref, out0_ref, w_ref, out1_ref, g2_ref):
    d = dinv_ref[...]
    t = -d * (a_ref[...] + b_ref[...])
    out1_ref[...] = out0_ref[...] + jnp.dot(
        t, w_ref[...], preferred_element_type=jnp.float32
    )
    g2_ref[...] = d * t


def _tc2(s1a, s1b, dinv, out0, w1):
    row = pl.BlockSpec((RB2, D), lambda i: (i, 0))
    return pl.pallas_call(
        _tc2_body,
        grid=(ACC_ROWS // RB2,),
        in_specs=[
            row,
            row,
            pl.BlockSpec((RB2, 1), lambda i: (i, 0)),
            row,
            pl.BlockSpec((D, D), lambda i: (0, 0)),
        ],
        out_specs=[row, row],
        out_shape=[
            jax.ShapeDtypeStruct((ACC_ROWS, D), jnp.float32),
            jax.ShapeDtypeStruct((ACC_ROWS, D), jnp.float32),
        ],
    )(s1a, s1b, dinv, out0, w1)


def _tc3_body(a_ref, b_ref, dinv_ref, x_ref, out1_ref, w_ref, bias_ref, y_ref):
    d = dinv_ref[...]
    xv = x_ref[...]
    tx2 = -2.0 * d * (a_ref[...] + b_ref[...]) - xv
    o = (
        out1_ref[...]
        + jnp.dot(tx2, w_ref[...], preferred_element_type=jnp.float32)
        + bias_ref[...]
    )
    y_ref[...] = jnp.maximum(o + xv, 0.0)


def _tc3(s2a, s2b, dinv, xp, out1, w2, bias):
    row = pl.BlockSpec((RB2, D), lambda i: (i, 0))
    return pl.pallas_call(
        _tc3_body,
        grid=(ACC_ROWS // RB2,),
        in_specs=[
            row,
            row,
            pl.BlockSpec((RB2, 1), lambda i: (i, 0)),
            row,
            row,
            pl.BlockSpec((D, D), lambda i: (0, 0)),
            pl.BlockSpec((1, D), lambda i: (0, 0)),
        ],
        out_specs=row,
        out_shape=jax.ShapeDtypeStruct((ACC_ROWS, D), jnp.float32),
    )(s2a, s2b, dinv, xp, out1, w2, bias)


def kernel(x, edge_index, W, b):
    x = x.astype(jnp.float32)
    row = edge_index[0].astype(jnp.int32)
    col = edge_index[1].astype(jnp.int32)
    npad = EPAD - E
    rpad = N + (jnp.arange(npad, dtype=jnp.int32) % (ACC_ROWS - N))
    cpad = jnp.arange(npad, dtype=jnp.int32) % (ACC_ROWS - N)
    rowp = jnp.concatenate([row, rpad])
    colp = jnp.concatenate([col, cpad])
    xp = jnp.concatenate([x, jnp.zeros((ACC_ROWS - N, D), jnp.float32)])

    out0 = _tc0(xp, W[0])
    degp = _sc_deg(rowp).reshape(NW, ACC_ROWS)
    dinv, g1 = _tc1(degp, xp)
    s1a, s1b = _sc_gather_scatter(g1, rowp, colp)
    out1, g2 = _tc2(s1a, s1b, dinv, out0, W[1])
    s2a, s2b = _sc_gather_scatter(g2, rowp, colp)
    yp = _tc3(s2a, s2b, dinv, xp, out1, W[2], b.reshape(1, D))
    return yp[:N]
